# Initial kernel scaffold; baseline (speedup 1.0000x reference)
import jax
import jax.numpy as jnp
from jax.experimental import pallas as pl


def kernel(peak_map, logit_thresh):
    raise NotImplementedError("write your pallas kernel here")


# speedup vs baseline: 4.9781x; 4.9781x over previous
import jax
import jax.numpy as jnp
from jax.experimental import pallas as pl
from jax.experimental.pallas import tpu as pltpu

_MD = 2
_SHARP = 10.0


def _sigmoid(z):
    return 1.0 / (1.0 + jnp.exp(-z))


def _body(lt_ref, x_ref, smooth_ref, mask_ref, pv_ref):
    x = x_ref[...]
    thresh = _sigmoid(lt_ref[0, 0])
    l1 = jnp.concatenate([x[:, :1], x[:, :-1]], axis=1)
    l2 = jnp.concatenate([x[:, :1], x[:, :1], x[:, :-2]], axis=1)
    r1 = jnp.concatenate([x[:, 1:], x[:, -1:]], axis=1)
    r2 = jnp.concatenate([x[:, 2:], x[:, -1:], x[:, -1:]], axis=1)
    pooled = jnp.maximum(jnp.maximum(jnp.maximum(l2, l1), jnp.maximum(r1, r2)), x)
    ea = jnp.exp(-_SHARP * (x - thresh))
    eb = jnp.exp(-_SHARP * (x - pooled))
    smooth = x / ((1.0 + ea) * (1.0 + eb))
    mask = smooth >= thresh
    smooth_ref[...] = smooth
    mask_ref[...] = mask
    pv_ref[...] = jnp.where(mask, x, 0.0)


def kernel(peak_map, logit_thresh):
    B, N = peak_map.shape
    lt = jnp.reshape(logit_thresh, (1, 1))
    return pl.pallas_call(
        _body,
        out_shape=(
            jax.ShapeDtypeStruct((B, N), jnp.float32),
            jax.ShapeDtypeStruct((B, N), jnp.bool_),
            jax.ShapeDtypeStruct((B, N), jnp.float32),
        ),
        in_specs=[
            pl.BlockSpec(memory_space=pltpu.SMEM),
            pl.BlockSpec(memory_space=pltpu.VMEM),
        ],
        out_specs=(
            pl.BlockSpec(memory_space=pltpu.VMEM),
            pl.BlockSpec(memory_space=pltpu.VMEM),
            pl.BlockSpec(memory_space=pltpu.VMEM),
        ),
    )(lt, peak_map)
